# baseline (speedup 1.0000x reference)
import functools

import jax
import jax.numpy as jnp
from jax import lax
from jax.experimental import pallas as pl
from jax.experimental.pallas import tpu as pltpu
from jax.experimental.pallas import tpu_sc as plsc

QDIM = 512
LDIM = 1024
HID = 128
EDIM = 16
NQ = 5000
NL = 5000
N = NQ + NL
E = 160000
ESEE = 120000
EPRED = 40000

F = 2 * HID
H = HID
EW = 128
C = 128
DUMP = N
NPAD = 10112
RPT = NPAD // 16
ZCH = ((0, 128), (128, 128), (256, 128), (384, 128), (512, RPT - 512))

K1 = 59
EP1 = 16 * K1 * C
K2 = 31
EP2 = 32 * K2 * C
C3 = 64
K3 = 21
EP3 = 32 * K3 * C3

_MESH = plsc.VectorSubcoreMesh(core_axis_name="c", subcore_axis_name="s")


def _zero_vmem(ref, rows, cols):
    z16 = jnp.zeros((16,), jnp.float32)

    def body(r, carry):
        for j in range(cols // 16):
            ref[r, pl.ds(j * 16, 16)] = z16
        return carry
    lax.fori_loop(0, rows, body, 0)


def _zero_acc(acc, rows_v, base):
    for off, ln in ZCH:
        pltpu.sync_copy(rows_v.at[pl.ds(0, ln)], acc.at[pl.ds(base + off, ln)])


def _drain_acc(acc, rows_v, base, out):
    for off, ln in ZCH:
        pltpu.sync_copy(acc.at[pl.ds(base + off, ln)], rows_v.at[pl.ds(0, ln)])
        pltpu.sync_copy(rows_v.at[pl.ds(0, ln)], out.at[pl.ds(base + off, ln)])


def _gather_scatter_pipelined(k, x_hbm, idx_v, dst_v, r0, r1, s0, s1, acc):
    assert k % 2 == 1 and k >= 3

    def gather(j, buf, sem):
        pltpu.async_copy(x_hbm.at[idx_v.at[j]], buf, sem)

    def wait(j, buf, sem):
        pltpu.make_async_copy(x_hbm.at[idx_v.at[j]], buf, sem).wait()

    def scat(j, buf):
        pltpu.sync_copy(buf, acc.at[dst_v.at[j]], add=True)

    gather(0, r0, s0)

    def pair(p, carry):
        j0 = 2 * p
        gather(j0 + 1, r1, s1)
        wait(j0, r0, s0)
        scat(j0, r0)
        gather(j0 + 2, r0, s0)
        wait(j0 + 1, r1, s1)
        scat(j0 + 1, r1)
        return carry
    lax.fori_loop(0, (k - 1) // 2, pair, 0)
    wait(k - 1, r0, s0)
    scat(k - 1, r0)


@functools.partial(
    pl.kernel,
    mesh=_MESH,
    out_type=[jax.ShapeDtypeStruct((NPAD, H), jnp.float32),
              jax.ShapeDtypeStruct((NPAD, H), jnp.float32)],
    scratch_types=[
        pltpu.VMEM((K1, C), jnp.int32),
        pltpu.VMEM((K1, C), jnp.int32),
        pltpu.VMEM((C, H), jnp.float32),
        pltpu.VMEM((C, H), jnp.float32),
        pltpu.VMEM_SHARED((NPAD, H), jnp.float32),
        pltpu.SemaphoreType.DMA,
        pltpu.SemaphoreType.DMA,
    ],
)
def _segsum_sc(x0, x1, srcs, dsts, out0, out1, src_v, dst_v, r0, r1, acc,
               s0, s1):
    cid = lax.axis_index("c")
    sid = lax.axis_index("s")
    base = sid * RPT

    _zero_vmem(r0, C, H)
    _zero_acc(acc, r0, base)
    pltpu.sync_copy(srcs.at[sid], src_v)
    pltpu.sync_copy(dsts.at[sid], dst_v)
    plsc.subcore_barrier()

    @pl.when(cid == 0)
    def _():
        _gather_scatter_pipelined(K1, x0, src_v, dst_v, r0, r1, s0, s1, acc)

    @pl.when(cid == 1)
    def _():
        _gather_scatter_pipelined(K1, x1, src_v, dst_v, r0, r1, s0, s1, acc)

    plsc.subcore_barrier()

    @pl.when(cid == 0)
    def _():
        _drain_acc(acc, r0, base, out0)

    @pl.when(cid == 1)
    def _():
        _drain_acc(acc, r0, base, out1)


@functools.partial(
    pl.kernel,
    mesh=_MESH,
    out_type=[jax.ShapeDtypeStruct((NPAD, EW), jnp.float32),
              jax.ShapeDtypeStruct((NPAD, EW), jnp.float32)],
    scratch_types=[
        pltpu.VMEM((K2, C), jnp.int32),
        pltpu.VMEM((K2, C), jnp.int32),
        pltpu.VMEM((C, EW), jnp.float32),
        pltpu.VMEM((C, EW), jnp.float32),
        pltpu.VMEM_SHARED((NPAD, EW), jnp.float32),
        pltpu.SemaphoreType.DMA,
        pltpu.SemaphoreType.DMA,
    ],
)
def _ewsum_sc(z, gidx, didx, out0, out1, g_v, d_v, r0, r1, acc, s0, s1):
    cid = lax.axis_index("c")
    sid = lax.axis_index("s")
    wid = cid * 16 + sid
    base = sid * RPT

    _zero_vmem(r0, C, EW)
    _zero_acc(acc, r0, base)
    pltpu.sync_copy(gidx.at[wid], g_v)
    pltpu.sync_copy(didx.at[wid], d_v)
    plsc.subcore_barrier()

    _gather_scatter_pipelined(K2, z, g_v, d_v, r0, r1, s0, s1, acc)

    plsc.subcore_barrier()

    @pl.when(cid == 0)
    def _():
        _drain_acc(acc, r0, base, out0)

    @pl.when(cid == 1)
    def _():
        _drain_acc(acc, r0, base, out1)


@functools.partial(
    pl.kernel,
    mesh=_MESH,
    out_type=jax.ShapeDtypeStruct((EP3, 16), jnp.float32),
    scratch_types=[
        pltpu.VMEM((K3, C3), jnp.int32),
        pltpu.VMEM((K3, C3), jnp.int32),
        pltpu.VMEM((C3, F), jnp.float32),
        pltpu.VMEM((C3, F), jnp.float32),
        pltpu.VMEM((C3, F), jnp.float32),
        pltpu.VMEM((C3, F), jnp.float32),
        pltpu.VMEM((C3, 16), jnp.float32),
        pltpu.SemaphoreType.DMA,
        pltpu.SemaphoreType.DMA,
    ],
)
def _pred_sc(xa, xb, aidx, bidx, out, ai_v, bi_v, a0, b0, a1, b1, res_v,
             s0, s1):
    cid = lax.axis_index("c")
    sid = lax.axis_index("s")
    wid = cid * 16 + sid

    pltpu.sync_copy(aidx.at[wid], ai_v)
    pltpu.sync_copy(bidx.at[wid], bi_v)

    def gather(j, a, b, sem):
        pltpu.async_copy(xa.at[ai_v.at[j]], a, sem)
        pltpu.async_copy(xb.at[bi_v.at[j]], b, sem)

    def wait(j, a, b, sem):
        pltpu.make_async_copy(xa.at[ai_v.at[j]], a, sem).wait()
        pltpu.make_async_copy(xb.at[bi_v.at[j]], b, sem).wait()

    def compute(j, a, b):
        def edge(e, c2):
            acc = a[e, pl.ds(0, 16)] * b[e, pl.ds(0, 16)]
            for k in range(1, F // 16):
                acc = acc + a[e, pl.ds(16 * k, 16)] * b[e, pl.ds(16 * k, 16)]
            res_v[e, :] = acc
            return c2
        lax.fori_loop(0, C3, edge, 0)
        pltpu.sync_copy(res_v, out.at[pl.ds(wid * (K3 * C3) + j * C3, C3)])

    gather(0, a0, b0, s0)

    def pair(p, carry):
        j0 = 2 * p
        gather(j0 + 1, a1, b1, s1)
        wait(j0, a0, b0, s0)
        compute(j0, a0, b0)
        gather(j0 + 2, a0, b0, s0)
        wait(j0 + 1, a1, b1, s1)
        compute(j0 + 1, a1, b1)
        return carry
    lax.fori_loop(0, (K3 - 1) // 2, pair, 0)
    wait(K3 - 1, a0, b0, s0)
    compute(K3 - 1, a0, b0)


def _mm_body(a_ref, w_ref, b_ref, o_ref):
    o_ref[...] = (jnp.dot(a_ref[...], w_ref[...],
                          preferred_element_type=jnp.float32) + b_ref[...])


def _matmul(a, w, b, bm):
    m, k = a.shape
    n = w.shape[1]
    return pl.pallas_call(
        _mm_body,
        grid=(m // bm,),
        in_specs=[pl.BlockSpec((bm, k), lambda i: (i, 0)),
                  pl.BlockSpec((k, n), lambda i: (0, 0)),
                  pl.BlockSpec((1, n), lambda i: (0, 0))],
        out_specs=pl.BlockSpec((bm, n), lambda i: (i, 0)),
        out_shape=jax.ShapeDtypeStruct((m, n), jnp.float32),
    )(a, w, b.reshape(1, n))


def _z_body(ew_ref, w_ref, b_ref, o_ref):
    t = (jnp.dot(ew_ref[...], w_ref[...],
                 preferred_element_type=jnp.float32) + b_ref[...])
    t = jnp.where(t >= 0, t, 0.01 * t)
    bm = t.shape[0]
    o_ref[...] = jnp.concatenate(
        [t, jnp.ones((bm, 1), jnp.float32),
         jnp.zeros((bm, EW - EDIM - 1), jnp.float32)], axis=1)


def _edge_z(edge_weight, we, be):
    bm = 1000
    return pl.pallas_call(
        _z_body,
        grid=(E // bm,),
        in_specs=[pl.BlockSpec((bm, EDIM), lambda i: (i, 0)),
                  pl.BlockSpec((EDIM, EDIM), lambda i: (0, 0)),
                  pl.BlockSpec((1, EDIM), lambda i: (0, 0))],
        out_specs=pl.BlockSpec((bm, EW), lambda i: (i, 0)),
        out_shape=jax.ShapeDtypeStruct((E, EW), jnp.float32),
    )(edge_weight, we, be.reshape(1, EDIM))


def _conv_body(s_ref, e0_ref, e1_ref, xp_ref, wm_ref, we_ref, y_ref, st_ref):
    y = jnp.dot(s_ref[...], wm_ref[...], preferred_element_type=jnp.float32)
    y = y + jnp.dot(e0_ref[...] + e1_ref[...], we_ref[...],
                    preferred_element_type=jnp.float32)
    y = y + xp_ref[...]
    y_ref[...] = y
    blk = jnp.concatenate(
        [jnp.sum(y, axis=0, keepdims=True),
         jnp.sum(y * y, axis=0, keepdims=True),
         jnp.zeros((6, F), jnp.float32)], axis=0)

    @pl.when(pl.program_id(0) == 0)
    def _():
        st_ref[...] = blk

    @pl.when(pl.program_id(0) > 0)
    def _():
        st_ref[...] = st_ref[...] + blk


def _conv_combine(s, e0, e1, xprev, wm, wed_aug):
    bm = 1000
    return pl.pallas_call(
        _conv_body,
        grid=(N // bm,),
        in_specs=[pl.BlockSpec((bm, F), lambda i: (i, 0)),
                  pl.BlockSpec((bm, EW), lambda i: (i, 0)),
                  pl.BlockSpec((bm, EW), lambda i: (i, 0)),
                  pl.BlockSpec((bm, F), lambda i: (i, 0)),
                  pl.BlockSpec((F, F), lambda i: (0, 0)),
                  pl.BlockSpec((EW, F), lambda i: (0, 0))],
        out_specs=[pl.BlockSpec((bm, F), lambda i: (i, 0)),
                   pl.BlockSpec((8, F), lambda i: (0, 0))],
        out_shape=[jax.ShapeDtypeStruct((N, F), jnp.float32),
                   jax.ShapeDtypeStruct((8, F), jnp.float32)],
    )(s, e0, e1, xprev, wm, wed_aug)


def _bn_body(y_ref, st_ref, g_ref, b_ref, o_ref, *, lrelu):
    mu = st_ref[0:1, :] * (1.0 / N)
    var = st_ref[1:2, :] * (1.0 / N) - mu * mu
    rs = lax.rsqrt(var + 1e-5)
    o = (y_ref[...] - mu) * (rs * g_ref[...]) + b_ref[...]
    if lrelu:
        o = jnp.where(o > 0, o, 0.01 * o)
    o_ref[...] = o


def _bn_apply(y, st, gamma, beta, lrelu):
    bm = 1000
    return pl.pallas_call(
        functools.partial(_bn_body, lrelu=lrelu),
        grid=(N // bm,),
        in_specs=[pl.BlockSpec((bm, F), lambda i: (i, 0)),
                  pl.BlockSpec((8, F), lambda i: (0, 0)),
                  pl.BlockSpec((1, F), lambda i: (0, 0)),
                  pl.BlockSpec((1, F), lambda i: (0, 0))],
        out_specs=pl.BlockSpec((bm, F), lambda i: (i, 0)),
        out_shape=jax.ShapeDtypeStruct((N, F), jnp.float32),
    )(y, st, gamma.reshape(1, F), beta.reshape(1, F))


def _predfin_body(p_ref, o_ref):
    s = jnp.sum(p_ref[...], axis=1) * (1.0 / F)
    o_ref[...] = 1.0 / (1.0 + jnp.exp(-s))


def _pred_finalize(p):
    bm = 3072
    return pl.pallas_call(
        _predfin_body,
        grid=(EP3 // bm,),
        in_specs=[pl.BlockSpec((bm, 16), lambda i: (i, 0))],
        out_specs=pl.BlockSpec((bm,), lambda i: (i,)),
        out_shape=jax.ShapeDtypeStruct((EP3,), jnp.float32),
    )(p)


def _pad_reshape(idx, total, ntiles, k, fill, c=C):
    pad = jnp.full((total - idx.shape[0],), fill, jnp.int32)
    return jnp.concatenate([idx, pad]).reshape(ntiles, k, c)


def kernel(task_id, query_features, llm_features, edge_weight, Wtask, btask,
           Wq, bq, Wllm, bllm, We_mlp, be_mlp, Wm1, bm1, Wed1, bed1, Wm2, bm2,
           Wed2, bed2, gamma1, beta1, gamma2, beta2, edge_index, edge_mask,
           edge_can_see):
    src_see = jnp.take(edge_index[0], edge_can_see)
    dst_see = jnp.take(edge_index[1], edge_can_see)
    sp = jnp.take(edge_index[0], edge_mask)
    dp = jnp.take(edge_index[1], edge_mask)

    srcs = _pad_reshape(src_see, EP1, 16, K1, 0)
    dsts = _pad_reshape(dst_see, EP1, 16, K1, DUMP)
    gidx = _pad_reshape(edge_can_see, EP2, 32, K2, 0)
    didx = _pad_reshape(dst_see, EP2, 32, K2, DUMP)
    aidx = _pad_reshape(sp, EP3, 32, K3, 0, C3)
    bidx = _pad_reshape(dp, EP3, 32, K3, 0, C3)

    wed1_aug = (jnp.zeros((EW, F), jnp.float32)
                .at[:EDIM].set(Wed1).at[EDIM].set(bm1 + bed1))
    wed2_aug = (jnp.zeros((EW, F), jnp.float32)
                .at[:EDIM].set(Wed2).at[EDIM].set(bm2 + bed2))

    at = _matmul(task_id, Wtask, btask, 1000)
    aq = _matmul(query_features, Wq, bq, 1000)
    al = _matmul(llm_features, Wllm, bllm, 1000)
    x_ini = jnp.concatenate([jnp.concatenate([at, aq], axis=1), al], axis=0)
    z = _edge_z(edge_weight, We_mlp, be_mlp)

    e0, e1 = _ewsum_sc(z, gidx, didx)
    e0 = e0[:N]
    e1 = e1[:N]

    s0, s1 = _segsum_sc(x_ini[:, :H], x_ini[:, H:], srcs, dsts)
    s = jnp.concatenate([s0[:N], s1[:N]], axis=1)
    y1, st1 = _conv_combine(s, e0, e1, x_ini, Wm1, wed1_aug)
    x1 = _bn_apply(y1, st1, gamma1, beta1, lrelu=True)

    s0, s1 = _segsum_sc(x1[:, :H], x1[:, H:], srcs, dsts)
    s = jnp.concatenate([s0[:N], s1[:N]], axis=1)
    y2, st2 = _conv_combine(s, e0, e1, x1, Wm2, wed2_aug)
    x2 = _bn_apply(y2, st2, gamma2, beta2, lrelu=False)

    p = _pred_sc(x_ini, x2, aidx, bidx)
    return _pred_finalize(p)[:EPRED]


# speedup vs baseline: 1.0825x; 1.0825x over previous
import functools

import jax
import jax.numpy as jnp
from jax import lax
from jax.experimental import pallas as pl
from jax.experimental.pallas import tpu as pltpu
from jax.experimental.pallas import tpu_sc as plsc

QDIM = 512
LDIM = 1024
HID = 128
EDIM = 16
NQ = 5000
NL = 5000
N = NQ + NL
E = 160000
ESEE = 120000
EPRED = 40000

F = 2 * HID
H = HID
EW = 128
C = 128
DUMP = N
NPAD = 10112
RPT = NPAD // 16
ZCH = ((0, 128), (128, 128), (256, 128), (384, 128), (512, RPT - 512))

K1 = 59
EP1 = 16 * K1 * C
K2 = 31
EP2 = 32 * K2 * C
C3 = 128
K3 = 10
EP3 = 32 * K3 * C3

_MESH = plsc.VectorSubcoreMesh(core_axis_name="c", subcore_axis_name="s")


def _zero_vmem(ref, rows, cols):
    z16 = jnp.zeros((16,), jnp.float32)

    def body(r, carry):
        for j in range(cols // 16):
            ref[r, pl.ds(j * 16, 16)] = z16
        return carry
    lax.fori_loop(0, rows, body, 0)


def _zero_acc(acc, rows_v, base):
    for off, ln in ZCH:
        pltpu.sync_copy(rows_v.at[pl.ds(0, ln)], acc.at[pl.ds(base + off, ln)])


def _drain_acc(acc, rows_v, base, out):
    for off, ln in ZCH:
        pltpu.sync_copy(acc.at[pl.ds(base + off, ln)], rows_v.at[pl.ds(0, ln)])
        pltpu.sync_copy(rows_v.at[pl.ds(0, ln)], out.at[pl.ds(base + off, ln)])


def _gather_scatter_pipelined(k, x_hbm, idx_v, dst_v, r0, r1, s0, s1, acc):
    assert k % 2 == 1 and k >= 3

    def gather(j, buf, sem):
        pltpu.async_copy(x_hbm.at[idx_v.at[j]], buf, sem)

    def wait(j, buf, sem):
        pltpu.make_async_copy(x_hbm.at[idx_v.at[j]], buf, sem).wait()

    def scat(j, buf):
        pltpu.sync_copy(buf, acc.at[dst_v.at[j]], add=True)

    gather(0, r0, s0)

    def pair(p, carry):
        j0 = 2 * p
        gather(j0 + 1, r1, s1)
        wait(j0, r0, s0)
        scat(j0, r0)
        gather(j0 + 2, r0, s0)
        wait(j0 + 1, r1, s1)
        scat(j0 + 1, r1)
        return carry
    lax.fori_loop(0, (k - 1) // 2, pair, 0)
    wait(k - 1, r0, s0)
    scat(k - 1, r0)


def _gather_scatter_serial(k, x_hbm, idx_v, dst_v, buf, sem, acc):
    def chunk(j, carry):
        pltpu.async_copy(x_hbm.at[idx_v.at[j]], buf, sem).wait()
        pltpu.sync_copy(buf, acc.at[dst_v.at[j]], add=True)
        return carry
    lax.fori_loop(0, k, chunk, 0)


@functools.partial(
    pl.kernel,
    mesh=_MESH,
    out_type=[jax.ShapeDtypeStruct((NPAD, H), jnp.float32),
              jax.ShapeDtypeStruct((NPAD, H), jnp.float32)],
    scratch_types=[
        pltpu.VMEM((K1, C), jnp.int32),
        pltpu.VMEM((K1, C), jnp.int32),
        pltpu.VMEM((C, H), jnp.float32),
        pltpu.VMEM((C, H), jnp.float32),
        pltpu.VMEM_SHARED((NPAD, H), jnp.float32),
        pltpu.SemaphoreType.DMA,
        pltpu.SemaphoreType.DMA,
    ],
)
def _segsum_sc(x0, x1, srcs, dsts, out0, out1, src_v, dst_v, r0, r1, acc,
               s0, s1):
    cid = lax.axis_index("c")
    sid = lax.axis_index("s")
    base = sid * RPT

    _zero_vmem(r0, C, H)
    _zero_acc(acc, r0, base)
    pltpu.sync_copy(srcs.at[sid], src_v)
    pltpu.sync_copy(dsts.at[sid], dst_v)
    plsc.subcore_barrier()

    @pl.when(cid == 0)
    def _():
        _gather_scatter_pipelined(K1, x0, src_v, dst_v, r0, r1, s0, s1, acc)

    @pl.when(cid == 1)
    def _():
        _gather_scatter_pipelined(K1, x1, src_v, dst_v, r0, r1, s0, s1, acc)

    plsc.subcore_barrier()

    @pl.when(cid == 0)
    def _():
        _drain_acc(acc, r0, base, out0)

    @pl.when(cid == 1)
    def _():
        _drain_acc(acc, r0, base, out1)


@functools.partial(
    pl.kernel,
    mesh=_MESH,
    out_type=[jax.ShapeDtypeStruct((NPAD, EW), jnp.float32),
              jax.ShapeDtypeStruct((NPAD, EW), jnp.float32)],
    scratch_types=[
        pltpu.VMEM((K2, C), jnp.int32),
        pltpu.VMEM((K2, C), jnp.int32),
        pltpu.VMEM((C, EW), jnp.float32),
        pltpu.VMEM_SHARED((NPAD, EW), jnp.float32),
        pltpu.SemaphoreType.DMA,
    ],
)
def _ewsum_sc(z, gidx, didx, out0, out1, g_v, d_v, r0, acc, s0):
    cid = lax.axis_index("c")
    sid = lax.axis_index("s")
    wid = cid * 16 + sid
    base = sid * RPT

    _zero_vmem(r0, C, EW)
    _zero_acc(acc, r0, base)
    pltpu.sync_copy(gidx.at[wid], g_v)
    pltpu.sync_copy(didx.at[wid], d_v)
    plsc.subcore_barrier()

    _gather_scatter_serial(K2, z, g_v, d_v, r0, s0, acc)

    plsc.subcore_barrier()

    @pl.when(cid == 0)
    def _():
        _drain_acc(acc, r0, base, out0)

    @pl.when(cid == 1)
    def _():
        _drain_acc(acc, r0, base, out1)


@functools.partial(
    pl.kernel,
    mesh=_MESH,
    out_type=jax.ShapeDtypeStruct((EP3, 16), jnp.float32),
    scratch_types=[
        pltpu.VMEM((K3, C3), jnp.int32),
        pltpu.VMEM((K3, C3), jnp.int32),
        pltpu.VMEM((C3, F), jnp.float32),
        pltpu.VMEM((C3, F), jnp.float32),
        pltpu.VMEM((C3, 16), jnp.float32),
        pltpu.SemaphoreType.DMA,
    ],
)
def _pred_sc(xa, xb, aidx, bidx, out, ai_v, bi_v, a_v, b_v, res_v, sem):
    cid = lax.axis_index("c")
    sid = lax.axis_index("s")
    wid = cid * 16 + sid

    pltpu.sync_copy(aidx.at[wid], ai_v)
    pltpu.sync_copy(bidx.at[wid], bi_v)

    def chunk(j, carry):
        ca = pltpu.async_copy(xa.at[ai_v.at[j]], a_v, sem)
        cb = pltpu.async_copy(xb.at[bi_v.at[j]], b_v, sem)
        ca.wait()
        cb.wait()

        def edge(e, c2):
            acc = a_v[e, pl.ds(0, 16)] * b_v[e, pl.ds(0, 16)]
            for k in range(1, F // 16):
                acc = acc + a_v[e, pl.ds(16 * k, 16)] * b_v[e, pl.ds(16 * k, 16)]
            res_v[e, :] = acc
            return c2
        lax.fori_loop(0, C3, edge, 0)
        pltpu.sync_copy(res_v, out.at[pl.ds(wid * (K3 * C3) + j * C3, C3)])
        return carry
    lax.fori_loop(0, K3, chunk, 0)


def _mm_body(a_ref, w_ref, b_ref, o_ref):
    o_ref[...] = (jnp.dot(a_ref[...], w_ref[...],
                          preferred_element_type=jnp.float32) + b_ref[...])


def _matmul(a, w, b, bm):
    m, k = a.shape
    n = w.shape[1]
    return pl.pallas_call(
        _mm_body,
        grid=(m // bm,),
        in_specs=[pl.BlockSpec((bm, k), lambda i: (i, 0)),
                  pl.BlockSpec((k, n), lambda i: (0, 0)),
                  pl.BlockSpec((1, n), lambda i: (0, 0))],
        out_specs=pl.BlockSpec((bm, n), lambda i: (i, 0)),
        out_shape=jax.ShapeDtypeStruct((m, n), jnp.float32),
    )(a, w, b.reshape(1, n))


def _z_body(ew_ref, w_ref, b_ref, o_ref):
    t = (jnp.dot(ew_ref[...], w_ref[...],
                 preferred_element_type=jnp.float32) + b_ref[...])
    t = jnp.where(t >= 0, t, 0.01 * t)
    bm = t.shape[0]
    o_ref[...] = jnp.concatenate(
        [t, jnp.ones((bm, 1), jnp.float32),
         jnp.zeros((bm, EW - EDIM - 1), jnp.float32)], axis=1)


def _edge_z(edge_weight, we, be):
    bm = 1000
    return pl.pallas_call(
        _z_body,
        grid=(E // bm,),
        in_specs=[pl.BlockSpec((bm, EDIM), lambda i: (i, 0)),
                  pl.BlockSpec((EDIM, EDIM), lambda i: (0, 0)),
                  pl.BlockSpec((1, EDIM), lambda i: (0, 0))],
        out_specs=pl.BlockSpec((bm, EW), lambda i: (i, 0)),
        out_shape=jax.ShapeDtypeStruct((E, EW), jnp.float32),
    )(edge_weight, we, be.reshape(1, EDIM))


def _conv_body(s_ref, e0_ref, e1_ref, xp_ref, wm_ref, we_ref, y_ref, st_ref):
    y = jnp.dot(s_ref[...], wm_ref[...], preferred_element_type=jnp.float32)
    y = y + jnp.dot(e0_ref[...] + e1_ref[...], we_ref[...],
                    preferred_element_type=jnp.float32)
    y = y + xp_ref[...]
    y_ref[...] = y
    blk = jnp.concatenate(
        [jnp.sum(y, axis=0, keepdims=True),
         jnp.sum(y * y, axis=0, keepdims=True),
         jnp.zeros((6, F), jnp.float32)], axis=0)

    @pl.when(pl.program_id(0) == 0)
    def _():
        st_ref[...] = blk

    @pl.when(pl.program_id(0) > 0)
    def _():
        st_ref[...] = st_ref[...] + blk


def _conv_combine(s, e0, e1, xprev, wm, wed_aug):
    bm = 1000
    return pl.pallas_call(
        _conv_body,
        grid=(N // bm,),
        in_specs=[pl.BlockSpec((bm, F), lambda i: (i, 0)),
                  pl.BlockSpec((bm, EW), lambda i: (i, 0)),
                  pl.BlockSpec((bm, EW), lambda i: (i, 0)),
                  pl.BlockSpec((bm, F), lambda i: (i, 0)),
                  pl.BlockSpec((F, F), lambda i: (0, 0)),
                  pl.BlockSpec((EW, F), lambda i: (0, 0))],
        out_specs=[pl.BlockSpec((bm, F), lambda i: (i, 0)),
                   pl.BlockSpec((8, F), lambda i: (0, 0))],
        out_shape=[jax.ShapeDtypeStruct((N, F), jnp.float32),
                   jax.ShapeDtypeStruct((8, F), jnp.float32)],
    )(s, e0, e1, xprev, wm, wed_aug)


def _bn_body(y_ref, st_ref, g_ref, b_ref, o_ref, *, lrelu):
    mu = st_ref[0:1, :] * (1.0 / N)
    var = st_ref[1:2, :] * (1.0 / N) - mu * mu
    rs = lax.rsqrt(var + 1e-5)
    o = (y_ref[...] - mu) * (rs * g_ref[...]) + b_ref[...]
    if lrelu:
        o = jnp.where(o > 0, o, 0.01 * o)
    o_ref[...] = o


def _bn_apply(y, st, gamma, beta, lrelu):
    bm = 1000
    return pl.pallas_call(
        functools.partial(_bn_body, lrelu=lrelu),
        grid=(N // bm,),
        in_specs=[pl.BlockSpec((bm, F), lambda i: (i, 0)),
                  pl.BlockSpec((8, F), lambda i: (0, 0)),
                  pl.BlockSpec((1, F), lambda i: (0, 0)),
                  pl.BlockSpec((1, F), lambda i: (0, 0))],
        out_specs=pl.BlockSpec((bm, F), lambda i: (i, 0)),
        out_shape=jax.ShapeDtypeStruct((N, F), jnp.float32),
    )(y, st, gamma.reshape(1, F), beta.reshape(1, F))


def _predfin_body(p_ref, o_ref):
    s = jnp.sum(p_ref[...], axis=1) * (1.0 / F)
    o_ref[...] = 1.0 / (1.0 + jnp.exp(-s))


def _pred_finalize(p):
    bm = 3072
    return pl.pallas_call(
        _predfin_body,
        grid=(EP3 // bm,),
        in_specs=[pl.BlockSpec((bm, 16), lambda i: (i, 0))],
        out_specs=pl.BlockSpec((bm,), lambda i: (i,)),
        out_shape=jax.ShapeDtypeStruct((EP3,), jnp.float32),
    )(p)


def _pad_reshape(idx, total, ntiles, k, fill, c=C):
    pad = jnp.full((total - idx.shape[0],), fill, jnp.int32)
    return jnp.concatenate([idx, pad]).reshape(ntiles, k, c)


def kernel(task_id, query_features, llm_features, edge_weight, Wtask, btask,
           Wq, bq, Wllm, bllm, We_mlp, be_mlp, Wm1, bm1, Wed1, bed1, Wm2, bm2,
           Wed2, bed2, gamma1, beta1, gamma2, beta2, edge_index, edge_mask,
           edge_can_see):
    src_see = jnp.take(edge_index[0], edge_can_see)
    dst_see = jnp.take(edge_index[1], edge_can_see)
    sp = jnp.take(edge_index[0], edge_mask)
    dp = jnp.take(edge_index[1], edge_mask)

    srcs = _pad_reshape(src_see, EP1, 16, K1, 0)
    dsts = _pad_reshape(dst_see, EP1, 16, K1, DUMP)
    gidx = _pad_reshape(edge_can_see, EP2, 32, K2, 0)
    didx = _pad_reshape(dst_see, EP2, 32, K2, DUMP)
    aidx = _pad_reshape(sp, EP3, 32, K3, 0, C3)
    bidx = _pad_reshape(dp, EP3, 32, K3, 0, C3)

    wed1_aug = (jnp.zeros((EW, F), jnp.float32)
                .at[:EDIM].set(Wed1).at[EDIM].set(bm1 + bed1))
    wed2_aug = (jnp.zeros((EW, F), jnp.float32)
                .at[:EDIM].set(Wed2).at[EDIM].set(bm2 + bed2))

    at = _matmul(task_id, Wtask, btask, 1000)
    aq = _matmul(query_features, Wq, bq, 1000)
    al = _matmul(llm_features, Wllm, bllm, 1000)
    x_ini = jnp.concatenate([jnp.concatenate([at, aq], axis=1), al], axis=0)
    z = _edge_z(edge_weight, We_mlp, be_mlp)

    e0, e1 = _ewsum_sc(z, gidx, didx)
    e0 = e0[:N]
    e1 = e1[:N]

    s0, s1 = _segsum_sc(x_ini[:, :H], x_ini[:, H:], srcs, dsts)
    s = jnp.concatenate([s0[:N], s1[:N]], axis=1)
    y1, st1 = _conv_combine(s, e0, e1, x_ini, Wm1, wed1_aug)
    x1 = _bn_apply(y1, st1, gamma1, beta1, lrelu=True)

    s0, s1 = _segsum_sc(x1[:, :H], x1[:, H:], srcs, dsts)
    s = jnp.concatenate([s0[:N], s1[:N]], axis=1)
    y2, st2 = _conv_combine(s, e0, e1, x1, Wm2, wed2_aug)
    x2 = _bn_apply(y2, st2, gamma2, beta2, lrelu=False)

    p = _pred_sc(x_ini, x2, aidx, bidx)
    return _pred_finalize(p)[:EPRED]


# speedup vs baseline: 1.1276x; 1.0416x over previous
import functools

import jax
import jax.numpy as jnp
from jax import lax
from jax.experimental import pallas as pl
from jax.experimental.pallas import tpu as pltpu
from jax.experimental.pallas import tpu_sc as plsc

QDIM = 512
LDIM = 1024
HID = 128
EDIM = 16
NQ = 5000
NL = 5000
N = NQ + NL
E = 160000
ESEE = 120000
EPRED = 40000

F = 2 * HID
H = HID
EW = 128
C = 128
DUMP = N
NPAD = 10112
RPT = NPAD // 16
ZCH = ((0, 128), (128, 128), (256, 128), (384, 128), (512, RPT - 512))

K1 = 59
EP1 = 16 * K1 * C
K2 = 31
EP2 = 32 * K2 * C
C3 = 128
K3 = 10
EP3 = 32 * K3 * C3

_MESH = plsc.VectorSubcoreMesh(core_axis_name="c", subcore_axis_name="s")


def _zero_vmem(ref, rows, cols):
    z16 = jnp.zeros((16,), jnp.float32)

    def body(r, carry):
        for j in range(cols // 16):
            ref[r, pl.ds(j * 16, 16)] = z16
        return carry
    lax.fori_loop(0, rows, body, 0)


def _zero_acc(acc, rows_v, base):
    for off, ln in ZCH:
        pltpu.sync_copy(rows_v.at[pl.ds(0, ln)], acc.at[pl.ds(base + off, ln)])


def _drain_acc(acc, rows_v, base, out, obase=0):
    for off, ln in ZCH:
        pltpu.sync_copy(acc.at[pl.ds(base + off, ln)], rows_v.at[pl.ds(0, ln)])
        pltpu.sync_copy(rows_v.at[pl.ds(0, ln)],
                        out.at[pl.ds(obase + base + off, ln)])


def _gather_scatter_pipelined(k, x_hbm, idx_v, dst_v, r0, r1, s0, s1, acc):
    assert k % 2 == 1 and k >= 3

    def gather(j, buf, sem):
        pltpu.async_copy(x_hbm.at[idx_v.at[j]], buf, sem)

    def wait(j, buf, sem):
        pltpu.make_async_copy(x_hbm.at[idx_v.at[j]], buf, sem).wait()

    def scat(j, buf):
        pltpu.sync_copy(buf, acc.at[dst_v.at[j]], add=True)

    gather(0, r0, s0)

    def pair(p, carry):
        j0 = 2 * p
        gather(j0 + 1, r1, s1)
        wait(j0, r0, s0)
        scat(j0, r0)
        gather(j0 + 2, r0, s0)
        wait(j0 + 1, r1, s1)
        scat(j0 + 1, r1)
        return carry
    lax.fori_loop(0, (k - 1) // 2, pair, 0)
    wait(k - 1, r0, s0)
    scat(k - 1, r0)


def _gather_scatter_serial(k, x_hbm, idx_v, dst_v, buf, sem, acc):
    def chunk(j, carry):
        pltpu.async_copy(x_hbm.at[idx_v.at[j]], buf, sem).wait()
        pltpu.sync_copy(buf, acc.at[dst_v.at[j]], add=True)
        return carry
    lax.fori_loop(0, k, chunk, 0)


@functools.partial(
    pl.kernel,
    mesh=_MESH,
    out_type=jax.ShapeDtypeStruct((2 * NPAD, H), jnp.float32),
    scratch_types=[
        pltpu.VMEM((K1, C), jnp.int32),
        pltpu.VMEM((K1, C), jnp.int32),
        pltpu.VMEM((C, H), jnp.float32),
        pltpu.VMEM((C, H), jnp.float32),
        pltpu.VMEM_SHARED((NPAD, H), jnp.float32),
        pltpu.SemaphoreType.DMA,
        pltpu.SemaphoreType.DMA,
    ],
)
def _segsum_sc(xv, srcs, dsts, out, src_v, dst_v, r0, r1, acc, s0, s1):
    cid = lax.axis_index("c")
    sid = lax.axis_index("s")
    base = sid * RPT

    _zero_vmem(r0, C, H)
    _zero_acc(acc, r0, base)
    pltpu.sync_copy(srcs.at[sid], src_v)
    pltpu.sync_copy(dsts.at[sid], dst_v)
    off = jnp.broadcast_to(cid * N, (16,)).astype(jnp.int32)

    def shift(r, carry):
        for j in range(C // 16):
            src_v[r, pl.ds(j * 16, 16)] = src_v[r, pl.ds(j * 16, 16)] + off
        return carry
    lax.fori_loop(0, K1, shift, 0)
    plsc.subcore_barrier()

    _gather_scatter_pipelined(K1, xv, src_v, dst_v, r0, r1, s0, s1, acc)

    plsc.subcore_barrier()
    _drain_acc(acc, r0, base, out, cid * NPAD)


@functools.partial(
    pl.kernel,
    mesh=_MESH,
    out_type=jax.ShapeDtypeStruct((2 * NPAD, EW), jnp.float32),
    scratch_types=[
        pltpu.VMEM((K2, C), jnp.int32),
        pltpu.VMEM((K2, C), jnp.int32),
        pltpu.VMEM((C, EW), jnp.float32),
        pltpu.VMEM_SHARED((NPAD, EW), jnp.float32),
        pltpu.SemaphoreType.DMA,
    ],
)
def _ewsum_sc(z, gidx, didx, out, g_v, d_v, r0, acc, s0):
    cid = lax.axis_index("c")
    sid = lax.axis_index("s")
    wid = cid * 16 + sid
    base = sid * RPT

    _zero_vmem(r0, C, EW)
    _zero_acc(acc, r0, base)
    pltpu.sync_copy(gidx.at[wid], g_v)
    pltpu.sync_copy(didx.at[wid], d_v)
    plsc.subcore_barrier()

    _gather_scatter_serial(K2, z, g_v, d_v, r0, s0, acc)

    plsc.subcore_barrier()
    _drain_acc(acc, r0, base, out, cid * NPAD)


@functools.partial(
    pl.kernel,
    mesh=_MESH,
    out_type=jax.ShapeDtypeStruct((EP3, 16), jnp.float32),
    scratch_types=[
        pltpu.VMEM((K3, C3), jnp.int32),
        pltpu.VMEM((K3, C3), jnp.int32),
        pltpu.VMEM((C3, F), jnp.float32),
        pltpu.VMEM((C3, F), jnp.float32),
        pltpu.VMEM((C3, 16), jnp.float32),
        pltpu.SemaphoreType.DMA,
    ],
)
def _pred_sc(xa, xb, aidx, bidx, out, ai_v, bi_v, a_v, b_v, res_v, sem):
    cid = lax.axis_index("c")
    sid = lax.axis_index("s")
    wid = cid * 16 + sid

    pltpu.sync_copy(aidx.at[wid], ai_v)
    pltpu.sync_copy(bidx.at[wid], bi_v)

    def chunk(j, carry):
        ca = pltpu.async_copy(xa.at[ai_v.at[j]], a_v, sem)
        cb = pltpu.async_copy(xb.at[bi_v.at[j]], b_v, sem)
        ca.wait()
        cb.wait()

        def edge(e, c2):
            acc = a_v[e, pl.ds(0, 16)] * b_v[e, pl.ds(0, 16)]
            for k in range(1, F // 16):
                acc = acc + a_v[e, pl.ds(16 * k, 16)] * b_v[e, pl.ds(16 * k, 16)]
            res_v[e, :] = acc
            return c2
        lax.fori_loop(0, C3, edge, 0)
        pltpu.sync_copy(res_v, out.at[pl.ds(wid * (K3 * C3) + j * C3, C3)])
        return carry
    lax.fori_loop(0, K3, chunk, 0)


def _mm_body(a_ref, w_ref, b_ref, o_ref):
    o_ref[...] = (jnp.dot(a_ref[...], w_ref[...],
                          preferred_element_type=jnp.float32) + b_ref[...])


def _matmul(a, w, b, bm):
    m, k = a.shape
    n = w.shape[1]
    return pl.pallas_call(
        _mm_body,
        grid=(m // bm,),
        in_specs=[pl.BlockSpec((bm, k), lambda i: (i, 0)),
                  pl.BlockSpec((k, n), lambda i: (0, 0)),
                  pl.BlockSpec((1, n), lambda i: (0, 0))],
        out_specs=pl.BlockSpec((bm, n), lambda i: (i, 0)),
        out_shape=jax.ShapeDtypeStruct((m, n), jnp.float32),
    )(a, w, b.reshape(1, n))


def _z_body(ew_ref, w_ref, b_ref, o_ref):
    t = (jnp.dot(ew_ref[...], w_ref[...],
                 preferred_element_type=jnp.float32) + b_ref[...])
    t = jnp.where(t >= 0, t, 0.01 * t)
    bm = t.shape[0]
    o_ref[...] = jnp.concatenate(
        [t, jnp.ones((bm, 1), jnp.float32),
         jnp.zeros((bm, EW - EDIM - 1), jnp.float32)], axis=1)


def _edge_z(edge_weight, we, be):
    bm = 1000
    return pl.pallas_call(
        _z_body,
        grid=(E // bm,),
        in_specs=[pl.BlockSpec((bm, EDIM), lambda i: (i, 0)),
                  pl.BlockSpec((EDIM, EDIM), lambda i: (0, 0)),
                  pl.BlockSpec((1, EDIM), lambda i: (0, 0))],
        out_specs=pl.BlockSpec((bm, EW), lambda i: (i, 0)),
        out_shape=jax.ShapeDtypeStruct((E, EW), jnp.float32),
    )(edge_weight, we, be.reshape(1, EDIM))


def _conv_body(s_ref, e0_ref, e1_ref, xp_ref, wm_ref, we_ref, y_ref, st_ref):
    y = jnp.dot(s_ref[...], wm_ref[...], preferred_element_type=jnp.float32)
    y = y + jnp.dot(e0_ref[...] + e1_ref[...], we_ref[...],
                    preferred_element_type=jnp.float32)
    y = y + xp_ref[...]
    y_ref[...] = y
    blk = jnp.concatenate(
        [jnp.sum(y, axis=0, keepdims=True),
         jnp.sum(y * y, axis=0, keepdims=True),
         jnp.zeros((6, F), jnp.float32)], axis=0)

    @pl.when(pl.program_id(0) == 0)
    def _():
        st_ref[...] = blk

    @pl.when(pl.program_id(0) > 0)
    def _():
        st_ref[...] = st_ref[...] + blk


def _conv_combine(s, e0, e1, xprev, wm, wed_aug):
    bm = 1000
    return pl.pallas_call(
        _conv_body,
        grid=(N // bm,),
        in_specs=[pl.BlockSpec((bm, F), lambda i: (i, 0)),
                  pl.BlockSpec((bm, EW), lambda i: (i, 0)),
                  pl.BlockSpec((bm, EW), lambda i: (i, 0)),
                  pl.BlockSpec((bm, F), lambda i: (i, 0)),
                  pl.BlockSpec((F, F), lambda i: (0, 0)),
                  pl.BlockSpec((EW, F), lambda i: (0, 0))],
        out_specs=[pl.BlockSpec((bm, F), lambda i: (i, 0)),
                   pl.BlockSpec((8, F), lambda i: (0, 0))],
        out_shape=[jax.ShapeDtypeStruct((N, F), jnp.float32),
                   jax.ShapeDtypeStruct((8, F), jnp.float32)],
    )(s, e0, e1, xprev, wm, wed_aug)


def _bn_body(y_ref, st_ref, g_ref, b_ref, o_ref, *, lrelu):
    mu = st_ref[0:1, :] * (1.0 / N)
    var = st_ref[1:2, :] * (1.0 / N) - mu * mu
    rs = lax.rsqrt(var + 1e-5)
    o = (y_ref[...] - mu) * (rs * g_ref[...]) + b_ref[...]
    if lrelu:
        o = jnp.where(o > 0, o, 0.01 * o)
    o_ref[...] = o


def _bn_apply(y, st, gamma, beta, lrelu):
    bm = 1000
    return pl.pallas_call(
        functools.partial(_bn_body, lrelu=lrelu),
        grid=(N // bm,),
        in_specs=[pl.BlockSpec((bm, F), lambda i: (i, 0)),
                  pl.BlockSpec((8, F), lambda i: (0, 0)),
                  pl.BlockSpec((1, F), lambda i: (0, 0)),
                  pl.BlockSpec((1, F), lambda i: (0, 0))],
        out_specs=pl.BlockSpec((bm, F), lambda i: (i, 0)),
        out_shape=jax.ShapeDtypeStruct((N, F), jnp.float32),
    )(y, st, gamma.reshape(1, F), beta.reshape(1, F))


def _predfin_body(p_ref, o_ref):
    s = jnp.sum(p_ref[...], axis=1) * (1.0 / F)
    o_ref[...] = 1.0 / (1.0 + jnp.exp(-s))


def _pred_finalize(p):
    bm = 4096
    assert EP3 % bm == 0
    return pl.pallas_call(
        _predfin_body,
        grid=(EP3 // bm,),
        in_specs=[pl.BlockSpec((bm, 16), lambda i: (i, 0))],
        out_specs=pl.BlockSpec((bm,), lambda i: (i,)),
        out_shape=jax.ShapeDtypeStruct((EP3,), jnp.float32),
    )(p)


def _pad_reshape(idx, total, ntiles, k, fill, c=C):
    pad = jnp.full((total - idx.shape[0],), fill, jnp.int32)
    return jnp.concatenate([idx, pad]).reshape(ntiles, k, c)


def kernel(task_id, query_features, llm_features, edge_weight, Wtask, btask,
           Wq, bq, Wllm, bllm, We_mlp, be_mlp, Wm1, bm1, Wed1, bed1, Wm2, bm2,
           Wed2, bed2, gamma1, beta1, gamma2, beta2, edge_index, edge_mask,
           edge_can_see):
    src_see = jnp.take(edge_index[0], edge_can_see)
    dst_see = jnp.take(edge_index[1], edge_can_see)
    sp = jnp.take(edge_index[0], edge_mask)
    dp = jnp.take(edge_index[1], edge_mask)

    srcs = _pad_reshape(src_see, EP1, 16, K1, 0)
    dsts = _pad_reshape(dst_see, EP1, 16, K1, DUMP)
    gidx = _pad_reshape(edge_can_see, EP2, 32, K2, 0)
    didx = _pad_reshape(dst_see, EP2, 32, K2, DUMP)
    aidx = _pad_reshape(sp, EP3, 32, K3, 0, C3)
    bidx = _pad_reshape(dp, EP3, 32, K3, 0, C3)

    wed1_aug = (jnp.zeros((EW, F), jnp.float32)
                .at[:EDIM].set(Wed1).at[EDIM].set(bm1 + bed1))
    wed2_aug = (jnp.zeros((EW, F), jnp.float32)
                .at[:EDIM].set(Wed2).at[EDIM].set(bm2 + bed2))

    at = _matmul(task_id, Wtask, btask, 1000)
    aq = _matmul(query_features, Wq, bq, 1000)
    al = _matmul(llm_features, Wllm, bllm, 1000)
    x_ini = jnp.concatenate([jnp.concatenate([at, aq], axis=1), al], axis=0)
    z = _edge_z(edge_weight, We_mlp, be_mlp)

    esum = _ewsum_sc(z, gidx, didx)
    e0 = esum[:N]
    e1 = esum[NPAD:NPAD + N]

    sv = _segsum_sc(jnp.concatenate([x_ini[:, :H], x_ini[:, H:]], 0),
                    srcs, dsts)
    s = jnp.concatenate([sv[:N], sv[NPAD:NPAD + N]], axis=1)
    y1, st1 = _conv_combine(s, e0, e1, x_ini, Wm1, wed1_aug)
    x1 = _bn_apply(y1, st1, gamma1, beta1, lrelu=True)

    sv = _segsum_sc(jnp.concatenate([x1[:, :H], x1[:, H:]], 0), srcs, dsts)
    s = jnp.concatenate([sv[:N], sv[NPAD:NPAD + N]], axis=1)
    y2, st2 = _conv_combine(s, e0, e1, x1, Wm2, wed2_aug)
    x2 = _bn_apply(y2, st2, gamma2, beta2, lrelu=False)

    p = _pred_sc(x_ini, x2, aidx, bidx)
    return _pred_finalize(p)[:EPRED]


# speedup vs baseline: 1.7960x; 1.5928x over previous
import functools

import jax
import jax.numpy as jnp
from jax import lax
from jax.experimental import pallas as pl
from jax.experimental.pallas import tpu as pltpu
from jax.experimental.pallas import tpu_sc as plsc

QDIM = 512
LDIM = 1024
HID = 128
EDIM = 16
NQ = 5000
NL = 5000
N = NQ + NL
E = 160000
ESEE = 120000
EPRED = 40000

F = 2 * HID
H = HID
EW = 128
C = 128
DUMP = N
NPAD = 10112
RPT = NPAD // 16
ZCH = ((0, 128), (128, 128), (256, 128), (384, 128), (512, RPT - 512))

K1 = 59
EP1 = 16 * K1 * C
K2 = 30
EP2 = 32 * K2 * C
C3 = 128
K3 = 10
EP3 = 32 * K3 * C3

_MESH = plsc.VectorSubcoreMesh(core_axis_name="c", subcore_axis_name="s")


def _zero_vmem(ref, rows, cols):
    z16 = jnp.zeros((16,), jnp.float32)

    def body(r, carry):
        for j in range(cols // 16):
            ref[r, pl.ds(j * 16, 16)] = z16
        return carry
    lax.fori_loop(0, rows, body, 0)


def _zero_acc(acc, rows_v, base):
    for off, ln in ZCH:
        pltpu.sync_copy(rows_v.at[pl.ds(0, ln)], acc.at[pl.ds(base + off, ln)])


def _drain_acc(acc, rows_v, base, out, obase=0):
    for off, ln in ZCH:
        pltpu.sync_copy(acc.at[pl.ds(base + off, ln)], rows_v.at[pl.ds(0, ln)])
        pltpu.sync_copy(rows_v.at[pl.ds(0, ln)],
                        out.at[pl.ds(obase + base + off, ln)])


def _gather_scatter_pipelined(k, x_hbm, idx_v, dst_v, r0, r1, s0, s1, acc):
    assert k % 2 == 1 and k >= 3

    def gather(j, buf, sem):
        pltpu.async_copy(x_hbm.at[idx_v.at[j]], buf, sem)

    def wait(j, buf, sem):
        pltpu.make_async_copy(x_hbm.at[idx_v.at[j]], buf, sem).wait()

    def scat(j, buf):
        pltpu.sync_copy(buf, acc.at[dst_v.at[j]], add=True)

    gather(0, r0, s0)

    def pair(p, carry):
        j0 = 2 * p
        gather(j0 + 1, r1, s1)
        wait(j0, r0, s0)
        scat(j0, r0)
        gather(j0 + 2, r0, s0)
        wait(j0 + 1, r1, s1)
        scat(j0 + 1, r1)
        return carry
    lax.fori_loop(0, (k - 1) // 2, pair, 0)
    wait(k - 1, r0, s0)
    scat(k - 1, r0)


def _gather_scatter_serial(k, x_hbm, idx_v, dst_v, buf, sem, acc):
    def chunk(j, carry):
        pltpu.async_copy(x_hbm.at[idx_v.at[j]], buf, sem).wait()
        pltpu.sync_copy(buf, acc.at[dst_v.at[j]], add=True)
        return carry
    lax.fori_loop(0, k, chunk, 0)


@functools.partial(
    pl.kernel,
    mesh=_MESH,
    out_type=jax.ShapeDtypeStruct((2 * NPAD, H), jnp.float32),
    scratch_types=[
        pltpu.VMEM((K1, C), jnp.int32),
        pltpu.VMEM((K1, C), jnp.int32),
        pltpu.VMEM((C, H), jnp.float32),
        pltpu.VMEM((C, H), jnp.float32),
        pltpu.VMEM_SHARED((NPAD, H), jnp.float32),
        pltpu.SemaphoreType.DMA,
        pltpu.SemaphoreType.DMA,
    ],
)
def _segsum_sc(xv, srcs, dsts, out, src_v, dst_v, r0, r1, acc, s0, s1):
    cid = lax.axis_index("c")
    sid = lax.axis_index("s")
    base = sid * RPT

    _zero_vmem(r0, C, H)
    _zero_acc(acc, r0, base)
    pltpu.sync_copy(srcs.at[sid], src_v)
    pltpu.sync_copy(dsts.at[sid], dst_v)
    off = jnp.broadcast_to(cid * N, (16,)).astype(jnp.int32)

    def shift(r, carry):
        for j in range(C // 16):
            src_v[r, pl.ds(j * 16, 16)] = src_v[r, pl.ds(j * 16, 16)] + off
        return carry
    lax.fori_loop(0, K1, shift, 0)
    plsc.subcore_barrier()

    _gather_scatter_pipelined(K1, xv, src_v, dst_v, r0, r1, s0, s1, acc)

    plsc.subcore_barrier()
    _drain_acc(acc, r0, base, out, cid * NPAD)


@functools.partial(
    pl.kernel,
    mesh=_MESH,
    out_type=jax.ShapeDtypeStruct((2 * NPAD, EW), jnp.float32),
    scratch_types=[
        pltpu.VMEM((K2, C), jnp.int32),
        pltpu.VMEM((K2, C), jnp.int32),
        pltpu.VMEM((C, EW), jnp.float32),
        pltpu.VMEM_SHARED((NPAD, EW), jnp.float32),
        pltpu.SemaphoreType.DMA,
    ],
)
def _ewsum_sc(z, gidx, didx, out, g_v, d_v, r0, acc, s0):
    cid = lax.axis_index("c")
    sid = lax.axis_index("s")
    wid = cid * 16 + sid
    base = sid * RPT

    _zero_vmem(r0, C, EW)
    _zero_acc(acc, r0, base)
    pltpu.sync_copy(gidx.at[wid], g_v)
    pltpu.sync_copy(didx.at[wid], d_v)
    plsc.subcore_barrier()

    _gather_scatter_serial(K2, z, g_v, d_v, r0, s0, acc)

    plsc.subcore_barrier()
    _drain_acc(acc, r0, base, out, cid * NPAD)


@functools.partial(
    pl.kernel,
    mesh=_MESH,
    out_type=jax.ShapeDtypeStruct((EP3, 16), jnp.float32),
    scratch_types=[
        pltpu.VMEM((K3, C3), jnp.int32),
        pltpu.VMEM((K3, C3), jnp.int32),
        pltpu.VMEM((C3, F), jnp.float32),
        pltpu.VMEM((C3, F), jnp.float32),
        pltpu.VMEM((C3, 16), jnp.float32),
        pltpu.SemaphoreType.DMA,
    ],
)
def _pred_sc(xa, xb, aidx, bidx, out, ai_v, bi_v, a_v, b_v, res_v, sem):
    cid = lax.axis_index("c")
    sid = lax.axis_index("s")
    wid = cid * 16 + sid

    pltpu.sync_copy(aidx.at[wid], ai_v)
    pltpu.sync_copy(bidx.at[wid], bi_v)

    def chunk(j, carry):
        ca = pltpu.async_copy(xa.at[ai_v.at[j]], a_v, sem)
        cb = pltpu.async_copy(xb.at[bi_v.at[j]], b_v, sem)
        ca.wait()
        cb.wait()

        def edge(e, c2):
            acc = a_v[e, pl.ds(0, 16)] * b_v[e, pl.ds(0, 16)]
            for k in range(1, F // 16):
                acc = acc + a_v[e, pl.ds(16 * k, 16)] * b_v[e, pl.ds(16 * k, 16)]
            res_v[e, :] = acc
            return c2
        lax.fori_loop(0, C3, edge, 0)
        pltpu.sync_copy(res_v, out.at[pl.ds(wid * (K3 * C3) + j * C3, C3)])
        return carry
    lax.fori_loop(0, K3, chunk, 0)


def _mm_body(a_ref, w_ref, b_ref, o_ref):
    o_ref[...] = (jnp.dot(a_ref[...], w_ref[...],
                          preferred_element_type=jnp.float32) + b_ref[...])


def _matmul(a, w, b, bm):
    m, k = a.shape
    n = w.shape[1]
    return pl.pallas_call(
        _mm_body,
        grid=(m // bm,),
        in_specs=[pl.BlockSpec((bm, k), lambda i: (i, 0)),
                  pl.BlockSpec((k, n), lambda i: (0, 0)),
                  pl.BlockSpec((1, n), lambda i: (0, 0))],
        out_specs=pl.BlockSpec((bm, n), lambda i: (i, 0)),
        out_shape=jax.ShapeDtypeStruct((m, n), jnp.float32),
    )(a, w, b.reshape(1, n))


def _z_body(ew_ref, w_ref, b_ref, o_ref):
    t = (jnp.dot(ew_ref[...], w_ref[...],
                 preferred_element_type=jnp.float32) + b_ref[...])
    t = jnp.where(t >= 0, t, 0.01 * t)
    bm = t.shape[0]
    o_ref[...] = jnp.concatenate(
        [t, jnp.ones((bm, 1), jnp.float32),
         jnp.zeros((bm, EW - EDIM - 1), jnp.float32)], axis=1)


def _edge_z(edge_weight, we, be):
    bm = 1000
    return pl.pallas_call(
        _z_body,
        grid=(E // bm,),
        in_specs=[pl.BlockSpec((bm, EDIM), lambda i: (i, 0)),
                  pl.BlockSpec((EDIM, EDIM), lambda i: (0, 0)),
                  pl.BlockSpec((1, EDIM), lambda i: (0, 0))],
        out_specs=pl.BlockSpec((bm, EW), lambda i: (i, 0)),
        out_shape=jax.ShapeDtypeStruct((E, EW), jnp.float32),
    )(edge_weight, we, be.reshape(1, EDIM))


def _conv_body(s_ref, e0_ref, e1_ref, xp_ref, wm_ref, we_ref, y_ref, st_ref):
    y = jnp.dot(s_ref[...], wm_ref[...], preferred_element_type=jnp.float32)
    y = y + jnp.dot(e0_ref[...] + e1_ref[...], we_ref[...],
                    preferred_element_type=jnp.float32)
    y = y + xp_ref[...]
    y_ref[...] = y
    blk = jnp.concatenate(
        [jnp.sum(y, axis=0, keepdims=True),
         jnp.sum(y * y, axis=0, keepdims=True),
         jnp.zeros((6, F), jnp.float32)], axis=0)

    @pl.when(pl.program_id(0) == 0)
    def _():
        st_ref[...] = blk

    @pl.when(pl.program_id(0) > 0)
    def _():
        st_ref[...] = st_ref[...] + blk


def _conv_combine(s, e0, e1, xprev, wm, wed_aug):
    bm = 1000
    return pl.pallas_call(
        _conv_body,
        grid=(N // bm,),
        in_specs=[pl.BlockSpec((bm, F), lambda i: (i, 0)),
                  pl.BlockSpec((bm, EW), lambda i: (i, 0)),
                  pl.BlockSpec((bm, EW), lambda i: (i, 0)),
                  pl.BlockSpec((bm, F), lambda i: (i, 0)),
                  pl.BlockSpec((F, F), lambda i: (0, 0)),
                  pl.BlockSpec((EW, F), lambda i: (0, 0))],
        out_specs=[pl.BlockSpec((bm, F), lambda i: (i, 0)),
                   pl.BlockSpec((8, F), lambda i: (0, 0))],
        out_shape=[jax.ShapeDtypeStruct((N, F), jnp.float32),
                   jax.ShapeDtypeStruct((8, F), jnp.float32)],
    )(s, e0, e1, xprev, wm, wed_aug)


def _bn_body(y_ref, st_ref, g_ref, b_ref, o_ref, *, lrelu):
    mu = st_ref[0:1, :] * (1.0 / N)
    var = st_ref[1:2, :] * (1.0 / N) - mu * mu
    rs = lax.rsqrt(var + 1e-5)
    o = (y_ref[...] - mu) * (rs * g_ref[...]) + b_ref[...]
    if lrelu:
        o = jnp.where(o > 0, o, 0.01 * o)
    o_ref[...] = o


def _bn_apply(y, st, gamma, beta, lrelu):
    bm = 1000
    return pl.pallas_call(
        functools.partial(_bn_body, lrelu=lrelu),
        grid=(N // bm,),
        in_specs=[pl.BlockSpec((bm, F), lambda i: (i, 0)),
                  pl.BlockSpec((8, F), lambda i: (0, 0)),
                  pl.BlockSpec((1, F), lambda i: (0, 0)),
                  pl.BlockSpec((1, F), lambda i: (0, 0))],
        out_specs=pl.BlockSpec((bm, F), lambda i: (i, 0)),
        out_shape=jax.ShapeDtypeStruct((N, F), jnp.float32),
    )(y, st, gamma.reshape(1, F), beta.reshape(1, F))


def _predfin_body(p_ref, o_ref):
    s = jnp.sum(p_ref[...], axis=1) * (1.0 / F)
    o_ref[...] = 1.0 / (1.0 + jnp.exp(-s))


def _pred_finalize(p):
    bm = 4096
    assert EP3 % bm == 0
    return pl.pallas_call(
        _predfin_body,
        grid=(EP3 // bm,),
        in_specs=[pl.BlockSpec((bm, 16), lambda i: (i, 0))],
        out_specs=pl.BlockSpec((bm,), lambda i: (i,)),
        out_shape=jax.ShapeDtypeStruct((EP3,), jnp.float32),
    )(p)


def _pad_reshape(idx, total, ntiles, k, fill, c=C):
    npad = total - idx.shape[0]
    if fill == DUMP:
        pad = N + jnp.arange(npad, dtype=jnp.int32) % (NPAD - N)
    else:
        pad = jnp.arange(npad, dtype=jnp.int32) % fill
    return jnp.concatenate([idx, pad]).reshape(ntiles, k, c)


def kernel(task_id, query_features, llm_features, edge_weight, Wtask, btask,
           Wq, bq, Wllm, bllm, We_mlp, be_mlp, Wm1, bm1, Wed1, bed1, Wm2, bm2,
           Wed2, bed2, gamma1, beta1, gamma2, beta2, edge_index, edge_mask,
           edge_can_see):
    src_see = jnp.take(edge_index[0], edge_can_see)
    dst_see = jnp.take(edge_index[1], edge_can_see)
    sp = jnp.take(edge_index[0], edge_mask)
    dp = jnp.take(edge_index[1], edge_mask)

    srcs = _pad_reshape(src_see, EP1, 16, K1, N)
    dsts = _pad_reshape(dst_see, EP1, 16, K1, DUMP)
    gidx = _pad_reshape(edge_can_see, EP2, 32, K2, E)
    didx = _pad_reshape(dst_see, EP2, 32, K2, DUMP)
    aidx = _pad_reshape(sp, EP3, 32, K3, N, C3)
    bidx = _pad_reshape(dp, EP3, 32, K3, N, C3)

    wed1_aug = (jnp.zeros((EW, F), jnp.float32)
                .at[:EDIM].set(Wed1).at[EDIM].set(bm1 + bed1))
    wed2_aug = (jnp.zeros((EW, F), jnp.float32)
                .at[:EDIM].set(Wed2).at[EDIM].set(bm2 + bed2))

    at = _matmul(task_id, Wtask, btask, 1000)
    aq = _matmul(query_features, Wq, bq, 1000)
    al = _matmul(llm_features, Wllm, bllm, 1000)
    x_ini = jnp.concatenate([jnp.concatenate([at, aq], axis=1), al], axis=0)
    z = _edge_z(edge_weight, We_mlp, be_mlp)

    esum = _ewsum_sc(z, gidx, didx)
    e0 = esum[:N]
    e1 = esum[NPAD:NPAD + N]

    sv = _segsum_sc(jnp.concatenate([x_ini[:, :H], x_ini[:, H:]], 0),
                    srcs, dsts)
    s = jnp.concatenate([sv[:N], sv[NPAD:NPAD + N]], axis=1)
    y1, st1 = _conv_combine(s, e0, e1, x_ini, Wm1, wed1_aug)
    x1 = _bn_apply(y1, st1, gamma1, beta1, lrelu=True)

    sv = _segsum_sc(jnp.concatenate([x1[:, :H], x1[:, H:]], 0), srcs, dsts)
    s = jnp.concatenate([sv[:N], sv[NPAD:NPAD + N]], axis=1)
    y2, st2 = _conv_combine(s, e0, e1, x1, Wm2, wed2_aug)
    x2 = _bn_apply(y2, st2, gamma2, beta2, lrelu=False)

    p = _pred_sc(x_ini, x2, aidx, bidx)
    return _pred_finalize(p)[:EPRED]
